# baseline (speedup 1.0000x reference)
import functools

import jax
import jax.numpy as jnp
from jax import lax
from jax.experimental import pallas as pl
from jax.experimental.pallas import tpu as pltpu
from jax.experimental.pallas import tpu_sc as plsc

EMB = 64
HID = 256
NC = 2
NS = 16
NW = NC * NS
CH = 128


def _sc_gather_body(uidx_hbm, ipair_hbm, uemb, ipair_emb, uout, iout,
                    uidx_v, ipr_v, ubuf, ibuf, semu, semv, semw, semx, semi,
                    *, bpw):
    wid = lax.axis_index("s") * NC + lax.axis_index("c")
    base = wid * bpw
    pltpu.sync_copy(uidx_hbm.at[pl.ds(base, bpw)], uidx_v)
    pltpu.sync_copy(ipair_hbm.at[pl.ds(base, bpw)], ipr_v)

    ci = pltpu.async_copy(ipair_emb.at[ipr_v], ibuf, semi)
    sems = (semu, semv, semw, semx)

    @pl.loop(0, bpw // CH)
    def _chunk(c):
        off = pl.multiple_of(c * CH, CH)
        for g in range(CH // 16):
            uv = uidx_v[pl.ds(off + g * 16, 16)]
            for lane in range(16):
                r = g * 16 + lane
                pltpu.async_copy(uemb.at[uv[lane]], ubuf.at[r],
                                 sems[g % len(sems)])
        q = CH // len(sems)
        for s in sems:
            pltpu.make_async_copy(uemb.at[pl.ds(0, q)],
                                  ubuf.at[pl.ds(0, q)], s).wait()
        pltpu.sync_copy(ubuf, uout.at[pl.ds(base + off, CH)])

    ci.wait()
    pltpu.sync_copy(ibuf, iout.at[pl.ds(base, bpw)])


def _mlp_body(gu, gip, isub, w1u, w1ii, b1, w2t, b2, o):
    half = lax.broadcasted_iota(jnp.int32, (1, 2 * EMB), 1) // EMB
    mask = (half == isub[...]).astype(jnp.float32)
    x = jnp.dot(gu[...], w1u[...], preferred_element_type=jnp.float32)
    x = x + jnp.dot(gip[...] * mask, w1ii[...],
                    preferred_element_type=jnp.float32)
    x = jnp.maximum(x + b1[...], 0.0)
    o[...] = jnp.sum(x * w2t[...], axis=1, keepdims=True) + b2[...]


def kernel(user_ids, item_ids, user_emb, item_emb, W1, b1, W2, b2):
    B = user_ids.shape[0]
    bpw = B // NW
    uids = user_ids.astype(jnp.int32)
    iids = item_ids.astype(jnp.int32)
    ipair_emb = item_emb.reshape(item_emb.shape[0] // 2, 2 * EMB)

    gather = pl.kernel(
        functools.partial(_sc_gather_body, bpw=bpw),
        out_type=(jax.ShapeDtypeStruct((B, EMB), jnp.float32),
                  jax.ShapeDtypeStruct((B, 2 * EMB), jnp.float32)),
        mesh=plsc.VectorSubcoreMesh(core_axis_name="c", subcore_axis_name="s"),
        scratch_types=[
            pltpu.VMEM((bpw,), jnp.int32),
            pltpu.VMEM((bpw,), jnp.int32),
            pltpu.VMEM((CH, EMB), jnp.float32),
            pltpu.VMEM((bpw, 2 * EMB), jnp.float32),
            pltpu.SemaphoreType.DMA,
            pltpu.SemaphoreType.DMA,
            pltpu.SemaphoreType.DMA,
            pltpu.SemaphoreType.DMA,
            pltpu.SemaphoreType.DMA,
        ],
    )
    gu, gip = gather(uids, iids >> 1, user_emb, ipair_emb)

    BM = 2048
    w1ii = jnp.concatenate([W1[EMB:], W1[EMB:]], axis=0)
    out = pl.pallas_call(
        _mlp_body,
        grid=(B // BM,),
        in_specs=[
            pl.BlockSpec((BM, EMB), lambda i: (i, 0)),
            pl.BlockSpec((BM, 2 * EMB), lambda i: (i, 0)),
            pl.BlockSpec((BM, 1), lambda i: (i, 0)),
            pl.BlockSpec((EMB, HID), lambda i: (0, 0)),
            pl.BlockSpec((2 * EMB, HID), lambda i: (0, 0)),
            pl.BlockSpec((1, HID), lambda i: (0, 0)),
            pl.BlockSpec((1, HID), lambda i: (0, 0)),
            pl.BlockSpec((1, 1), lambda i: (0, 0)),
        ],
        out_specs=pl.BlockSpec((BM, 1), lambda i: (i, 0)),
        out_shape=jax.ShapeDtypeStruct((B, 1), jnp.float32),
    )(gu, gip, (iids & 1).reshape(B, 1),
      W1[:EMB], w1ii, b1.reshape(1, HID),
      W2.reshape(1, HID), b2.reshape(1, 1))
    return out


# speedup vs baseline: 1.0568x; 1.0568x over previous
import functools

import jax
import jax.numpy as jnp
from jax import lax
from jax.experimental import pallas as pl
from jax.experimental.pallas import tpu as pltpu
from jax.experimental.pallas import tpu_sc as plsc

EMB = 64
HID = 256
NC = 2
NS = 16
NW = NC * NS
CH = 128


def _sc_gather_body(uidx_hbm, iidx_hbm, uemb, iemb, uout, iout,
                    uidx_v, iidx_v, ubuf, ibuf, sem, *, bpw):
    wid = lax.axis_index("s") * NC + lax.axis_index("c")
    base = wid * bpw
    pltpu.sync_copy(uidx_hbm.at[pl.ds(base, bpw)], uidx_v)
    pltpu.sync_copy(iidx_hbm.at[pl.ds(base, bpw)], iidx_v)

    @pl.loop(0, bpw // CH)
    def _chunk(c):
        off = pl.multiple_of(c * CH, CH)
        for g in range(CH // 16):
            uv = uidx_v[pl.ds(off + g * 16, 16)]
            iv = iidx_v[pl.ds(off + g * 16, 16)]
            for lane in range(16):
                r = g * 16 + lane
                pltpu.async_copy(uemb.at[uv[lane]], ubuf.at[r], sem)
                pltpu.async_copy(iemb.at[iv[lane]], ibuf.at[r], sem)
        pltpu.make_async_copy(uemb.at[pl.ds(0, CH)], ubuf, sem).wait()
        pltpu.make_async_copy(iemb.at[pl.ds(0, CH)], ibuf, sem).wait()
        pltpu.sync_copy(ubuf, uout.at[pl.ds(base + off, CH)])
        pltpu.sync_copy(ibuf, iout.at[pl.ds(base + off, CH)])


def _mlp_body(gu, gi, w1u, w1i, b1, w2t, b2, o):
    x = jnp.dot(gu[...], w1u[...], preferred_element_type=jnp.float32)
    x = x + jnp.dot(gi[...], w1i[...], preferred_element_type=jnp.float32)
    x = jnp.maximum(x + b1[...], 0.0)
    o[...] = jnp.sum(x * w2t[...], axis=1, keepdims=True) + b2[...]


def kernel(user_ids, item_ids, user_emb, item_emb, W1, b1, W2, b2):
    B = user_ids.shape[0]
    bpw = B // NW
    uidx = user_ids.astype(jnp.int32)
    iidx = item_ids.astype(jnp.int32)

    gather = pl.kernel(
        functools.partial(_sc_gather_body, bpw=bpw),
        out_type=(jax.ShapeDtypeStruct((B, EMB), jnp.float32),
                  jax.ShapeDtypeStruct((B, EMB), jnp.float32)),
        mesh=plsc.VectorSubcoreMesh(core_axis_name="c", subcore_axis_name="s"),
        scratch_types=[
            pltpu.VMEM((bpw,), jnp.int32),
            pltpu.VMEM((bpw,), jnp.int32),
            pltpu.VMEM((CH, EMB), jnp.float32),
            pltpu.VMEM((CH, EMB), jnp.float32),
            pltpu.SemaphoreType.DMA,
        ],
    )
    gu, gi = gather(uidx, iidx, user_emb, item_emb)

    BM = 2048
    out = pl.pallas_call(
        _mlp_body,
        grid=(B // BM,),
        in_specs=[
            pl.BlockSpec((BM, EMB), lambda i: (i, 0)),
            pl.BlockSpec((BM, EMB), lambda i: (i, 0)),
            pl.BlockSpec((EMB, HID), lambda i: (0, 0)),
            pl.BlockSpec((EMB, HID), lambda i: (0, 0)),
            pl.BlockSpec((1, HID), lambda i: (0, 0)),
            pl.BlockSpec((1, HID), lambda i: (0, 0)),
            pl.BlockSpec((1, 1), lambda i: (0, 0)),
        ],
        out_specs=pl.BlockSpec((BM, 1), lambda i: (i, 0)),
        out_shape=jax.ShapeDtypeStruct((B, 1), jnp.float32),
    )(gu, gi, W1[:EMB], W1[EMB:], b1.reshape(1, HID),
      W2.reshape(1, HID), b2.reshape(1, 1))
    return out
